# Initial kernel scaffold; baseline (speedup 1.0000x reference)
import jax
import jax.numpy as jnp
from jax.experimental import pallas as pl


def kernel(features, nodes, to_neighs, num_sample):
    raise NotImplementedError("write your pallas kernel here")


# speedup vs baseline: 1.8616x; 1.8616x over previous
import functools

import jax
import jax.numpy as jnp
from jax import lax
from jax.experimental import pallas as pl
from jax.experimental.pallas import tpu as pltpu
from jax.experimental.pallas import tpu_sc as plsc

L = 16
NC = 2
NS = 16
NW = NC * NS
G = 4


def _mean_agg(features, idx_flat, *, B_pad, S, D):
    C = B_pad // NW
    n_groups = C // G
    scale = jnp.float32(1.0 / S)

    mesh = plsc.VectorSubcoreMesh(
        core_axis_name="c", subcore_axis_name="s",
        num_cores=NC, num_subcores=NS,
    )

    @functools.partial(
        pl.kernel,
        out_type=jax.ShapeDtypeStruct((B_pad * D,), jnp.float32),
        mesh=mesh,
        scratch_types=[
            pltpu.VMEM((G * S,), jnp.int32),
            pltpu.VMEM((G * S, D), jnp.float32),
            pltpu.VMEM((G * D,), jnp.float32),
            pltpu.SemaphoreType.DMA,
        ],
    )
    def k(feat_hbm, idx_hbm, out_hbm, idx_v, rows_v, acc_v, sem):
        cid = lax.axis_index("c")
        sid = lax.axis_index("s")
        wid = sid * NC + cid
        base = wid * C

        def group(g, carry):
            node0 = base + g * G
            pltpu.sync_copy(idx_hbm.at[pl.ds(node0 * S, G * S)], idx_v)
            pltpu.async_copy(feat_hbm.at[idx_v], rows_v, sem).wait()
            for i in range(G):
                def srow(s, acc):
                    return tuple(
                        acc[l] + rows_v[i * S + s, pl.ds(l * L, L)]
                        for l in range(D // L)
                    )
                acc = lax.fori_loop(
                    1, S, srow,
                    tuple(rows_v[i * S, pl.ds(l * L, L)] for l in range(D // L)),
                )
                for l in range(D // L):
                    acc_v[pl.ds(i * D + l * L, L)] = acc[l] * scale
            pltpu.sync_copy(acc_v, out_hbm.at[pl.ds(node0 * D, G * D)])
            return carry

        lax.fori_loop(0, n_groups, group, 0)

    return k(features, idx_flat)


def kernel(features, nodes, to_neighs, num_sample):
    B, S = to_neighs.shape
    N, D = features.shape
    chunk = NW * G
    B_pad = ((B + chunk - 1) // chunk) * chunk
    tn = to_neighs.astype(jnp.int32)
    if B_pad != B:
        tn = jnp.pad(tn, ((0, B_pad - B), (0, 0)))
    idx_flat = tn.reshape(-1)
    out = _mean_agg(features, idx_flat, B_pad=B_pad, S=S, D=D)
    return out.reshape(B_pad, D)[:B]
